# baseline (speedup 1.0000x reference)
import numpy as np
import jax
import jax.numpy as jnp
from jax import lax
from jax.experimental import pallas as pl
from jax.experimental.pallas import tpu as pltpu

_BLK1 = 1000
_BLK2 = 5000
_N_BANK = 100000
_NB1 = _N_BANK // _BLK1
_NB2 = _N_BANK // _BLK2
_RING = 4
_C = 64
_K = 9
_N_PIX = 784
_HW = 196
_BSZ = 4
_NCAND = 16
_BIG = 3.0e38


def _resize_mat(inp=14, out=224):
    scale = inp / out
    x = (np.arange(out) + 0.5) * scale - 0.5
    j = np.arange(inp)
    w = np.maximum(0.0, 1.0 - np.abs(x[:, None] - j[None, :]))
    return w / w.sum(axis=1, keepdims=True)


def _blur_mat(n=224, sigma=4.0):
    r = int(round(4 * sigma))
    size = 2 * r + 1
    ax = np.arange(size) - r
    g = np.exp(-(ax * ax) / (2.0 * sigma * sigma))
    g = g / g.sum()
    G = np.zeros((n, n), np.float64)
    for i in range(n):
        lo = max(0, i - r)
        hi = min(n, i + r + 1)
        G[i, lo:hi] = g[(lo - i) + r:(hi - i) + r]
    return G


_A_MAT = np.ascontiguousarray((_blur_mat() @ _resize_mat()).astype(np.float32))
_AT_MAT = np.ascontiguousarray(_A_MAT.T)


def _stage_copy(b_hbm, ring_ref, sem_ref, j):
    src = pl.ds(j * _BLK1, _BLK1)
    dst = pl.ds((j % _RING) * _BLK1, _BLK1)
    return pltpu.make_async_copy(b_hbm.at[src, :], ring_ref.at[dst, :],
                                 sem_ref.at[j % _RING])


def _aug(b):
    bb = jnp.sum(b * b, axis=1, keepdims=True)
    bbh = bb.astype(jnp.bfloat16)
    bbl = (bb - bbh.astype(jnp.float32)).astype(jnp.bfloat16)
    return jnp.concatenate([b.astype(jnp.bfloat16), bbh, bbl], axis=1)


def _mega_kern(at_ref, a2_ref, aa_ref, b_hbm, mine_ref, s_ref,
               bank_ref, ring_ref, rrows_ref, top_ref, nn_ref,
               sems, rsems):
    at_bf = at_ref[...].astype(jnp.bfloat16)

    for j in range(_RING - 1):
        _stage_copy(b_hbm, ring_ref, sems, j).start()

    def p1_body(i, acc):
        _stage_copy(b_hbm, ring_ref, sems, i).wait()
        b = ring_ref[pl.ds((i % _RING) * _BLK1, _BLK1), :]
        b_aug = _aug(b)
        bank_ref[pl.ds(i * _BLK1, _BLK1), :] = b_aug
        e = lax.dot_general(b_aug, at_bf, (((1,), (0,)), ((), ())),
                            preferred_element_type=jnp.float32)
        acc = jnp.minimum(acc, jnp.min(e, axis=0, keepdims=True))

        @pl.when(i + _RING - 1 < _NB1)
        def _():
            _stage_copy(b_hbm, ring_ref, sems, i + _RING - 1).start()
        return acc

    min_e = lax.fori_loop(0, _NB1, p1_body,
                          jnp.full((1, _N_PIX), _BIG, jnp.float32))
    mine_ref[...] = min_e

    d2all = min_e + aa_ref[...]
    lane = lax.broadcasted_iota(jnp.int32, (1, _N_PIX), 1)
    batch_id = lane // _HW
    cand_idx = []
    for b in range(_BSZ):
        dm = jnp.where(batch_id == b, d2all, -_BIG)
        for _ in range(4):
            am = jnp.argmax(dm[0], axis=0)
            cand_idx.append(am)
            dm = jnp.where(lane == am, -_BIG, dm)

    cand_rows = [a2_ref[pl.ds(ix, 1), :] for ix in cand_idx]
    candT = jnp.concatenate(cand_rows, axis=0)
    cand_bf = candT.astype(jnp.bfloat16)
    aa_list = [jnp.sum(jnp.where(lane == ix, aa_ref[...], 0.0), axis=1,
                       keepdims=True) for ix in cand_idx]
    aa16 = jnp.concatenate(aa_list, axis=1)

    top_ref[...] = jnp.full((_NCAND, _NCAND), _BIG, jnp.float32)
    nn_ref[...] = jnp.zeros((1, _NCAND), jnp.int32)
    lane2 = lax.broadcasted_iota(jnp.int32, (1, _NCAND), 1)

    def p2_body(i, curmin):
        b_aug = bank_ref[pl.ds(i * _BLK2, _BLK2), :]
        e2 = lax.dot_general(b_aug, cand_bf, (((1,), (1,)), ((), ())),
                             preferred_element_type=jnp.float32)
        m = jnp.min(e2, axis=0, keepdims=True)
        am = jnp.argmin(e2, axis=0)[None, :] + i * _BLK2
        better = m < curmin
        nn_ref[...] = jnp.where(better, am, nn_ref[...])

        @pl.when(jnp.any(m < top_ref[_K - 1:_K, :]))
        def _():
            comb = jnp.concatenate([top_ref[...], e2], axis=0)
            srow = lax.broadcasted_iota(jnp.int32, comb.shape, 0)
            rows = []
            for _ in range(_K):
                v = jnp.min(comb, axis=0, keepdims=True)
                amr = jnp.argmin(comb, axis=0)[None, :]
                comb = jnp.where(srow == amr, _BIG, comb)
                rows.append(v)
            rows.append(jnp.full((_NCAND - _K, _NCAND), _BIG, jnp.float32))
            top_ref[...] = jnp.concatenate(rows, axis=0)

        return jnp.minimum(curmin, m)

    lax.fori_loop(0, _NB2, p2_body, jnp.full((1, _NCAND), _BIG, jnp.float32))

    for cpos in range(_NCAND):
        row = nn_ref[0, cpos]
        pltpu.make_async_copy(b_hbm.at[pl.ds(row, 1), :],
                              rrows_ref.at[pl.ds(cpos, 1), :],
                              rsems.at[cpos]).start()
    for cpos in range(_NCAND):
        row = nn_ref[0, cpos]
        pltpu.make_async_copy(b_hbm.at[pl.ds(row, 1), :],
                              rrows_ref.at[pl.ds(cpos, 1), :],
                              rsems.at[cpos]).wait()
    d2x_cols = []
    for cpos in range(_NCAND):
        brow = rrows_ref[cpos:cpos + 1, :]
        arow = candT[cpos:cpos + 1, :_C]
        e_exact = jnp.sum(arow * brow + brow * brow, axis=1, keepdims=True)
        d2x_cols.append(e_exact)
    d2x = jnp.concatenate(d2x_cols, axis=1) + aa16

    conf0 = jnp.sqrt(jnp.maximum(d2x, 1e-12))
    conf_rest = jnp.sqrt(jnp.maximum(top_ref[1:_K, :] + aa16, 1e-12))
    conf = jnp.concatenate([conf0, conf_rest], axis=0)
    ec = jnp.exp(conf)
    wgt = 1.0 - jnp.max(ec, axis=0, keepdims=True) / jnp.sum(ec, axis=0, keepdims=True)
    s16 = conf0 * wgt

    grp = lane2 // 4
    s_rows = []
    for b in range(_BSZ):
        db = jnp.where(grp == b, d2x, -_BIG)
        vb = jnp.max(db)
        sb = jnp.max(jnp.where((grp == b) & (db == vb), s16, -_BIG))
        s_rows.append(jnp.full((1, 1), 1.0, jnp.float32) * sb)
    s_rows.append(jnp.zeros((4, 1), jnp.float32))
    s_ref[...] = jnp.concatenate(s_rows, axis=0)


def _mask_kern(d2_ref, a_ref, at_ref, o_ref):
    m14 = jnp.sqrt(jnp.maximum(d2_ref[0], 1e-12))
    t = jnp.dot(a_ref[...], m14, preferred_element_type=jnp.float32)
    o_ref[0] = jnp.dot(t, at_ref[...], preferred_element_type=jnp.float32)


def kernel(inputs, feature_vector):
    bsz, h, w, c = inputs.shape
    n_pix = bsz * h * w
    q = inputs.reshape(n_pix, c)
    aa = jnp.sum(q * q, axis=1)[None, :]
    a2 = jnp.concatenate([-2.0 * q, jnp.ones((n_pix, 2), jnp.float32)], axis=1)
    a_augT = a2.T

    min_e, s8 = pl.pallas_call(
        _mega_kern,
        in_specs=[
            pl.BlockSpec((c + 2, n_pix), lambda: (0, 0)),
            pl.BlockSpec((n_pix, c + 2), lambda: (0, 0)),
            pl.BlockSpec((1, n_pix), lambda: (0, 0)),
            pl.BlockSpec(memory_space=pl.ANY),
        ],
        out_specs=[
            pl.BlockSpec((1, n_pix), lambda: (0, 0)),
            pl.BlockSpec((8, 1), lambda: (0, 0)),
        ],
        out_shape=[
            jax.ShapeDtypeStruct((1, n_pix), jnp.float32),
            jax.ShapeDtypeStruct((8, 1), jnp.float32),
        ],
        scratch_shapes=[
            pltpu.VMEM((_N_BANK, _C + 2), jnp.bfloat16),
            pltpu.VMEM((_RING * _BLK1, _C), jnp.float32),
            pltpu.VMEM((_NCAND, _C), jnp.float32),
            pltpu.VMEM((_NCAND, _NCAND), jnp.float32),
            pltpu.VMEM((1, _NCAND), jnp.int32),
            pltpu.SemaphoreType.DMA((_RING,)),
            pltpu.SemaphoreType.DMA((_NCAND,)),
        ],
    )(a_augT, a2, aa, feature_vector)

    s = s8[:bsz]
    d2min = (aa + min_e).reshape(bsz, h, w)

    mask = pl.pallas_call(
        _mask_kern,
        grid=(bsz,),
        in_specs=[
            pl.BlockSpec((1, h, w), lambda i: (i, 0, 0)),
            pl.BlockSpec((224, h), lambda i: (0, 0)),
            pl.BlockSpec((h, 224), lambda i: (0, 0)),
        ],
        out_specs=pl.BlockSpec((1, 224, 224), lambda i: (i, 0, 0)),
        out_shape=jax.ShapeDtypeStruct((bsz, 224, 224), jnp.float32),
    )(d2min, jnp.asarray(_A_MAT), jnp.asarray(_AT_MAT))

    return (s, mask.reshape(bsz, 224, 224, 1))


# speedup vs baseline: 1.0832x; 1.0832x over previous
import numpy as np
import jax
import jax.numpy as jnp
from jax import lax
from jax.experimental import pallas as pl
from jax.experimental.pallas import tpu as pltpu

_BLK1 = 4000
_BLK2 = 5000
_N_BANK = 100000
_NB1 = _N_BANK // _BLK1
_NB2 = _N_BANK // _BLK2
_C = 64
_K = 9
_N_PIX = 784
_HW = 196
_BSZ = 4
_NCAND = 32
_NPB = _NCAND // _BSZ
_BIG = 3.0e38


def _resize_mat(inp=14, out=224):
    scale = inp / out
    x = (np.arange(out) + 0.5) * scale - 0.5
    j = np.arange(inp)
    w = np.maximum(0.0, 1.0 - np.abs(x[:, None] - j[None, :]))
    return w / w.sum(axis=1, keepdims=True)


def _blur_mat(n=224, sigma=4.0):
    r = int(round(4 * sigma))
    size = 2 * r + 1
    ax = np.arange(size) - r
    g = np.exp(-(ax * ax) / (2.0 * sigma * sigma))
    g = g / g.sum()
    G = np.zeros((n, n), np.float64)
    for i in range(n):
        lo = max(0, i - r)
        hi = min(n, i + r + 1)
        G[i, lo:hi] = g[(lo - i) + r:(hi - i) + r]
    return G


_A_MAT = np.ascontiguousarray((_blur_mat() @ _resize_mat()).astype(np.float32))
_AT_MAT = np.ascontiguousarray(_A_MAT.T)


def _aug(b):
    bb = jnp.sum(b * b, axis=1, keepdims=True)
    bbh = bb.astype(jnp.bfloat16)
    bbl = (bb - bbh.astype(jnp.float32)).astype(jnp.bfloat16)
    return jnp.concatenate([b.astype(jnp.bfloat16), bbh, bbl], axis=1)


def _mega_kern(at_ref, a2_ref, aa_ref, b_blk_ref, b_hbm, mine_ref, s_ref,
               bank_ref, rrows_ref, acc_ref, rsems):
    i = pl.program_id(0)

    b_aug = _aug(b_blk_ref[...])
    bank_ref[pl.ds(i * _BLK1, _BLK1), :] = b_aug
    e = lax.dot_general(b_aug, at_ref[...].astype(jnp.bfloat16),
                        (((1,), (0,)), ((), ())),
                        preferred_element_type=jnp.float32)
    m = jnp.min(e, axis=0, keepdims=True)

    @pl.when(i == 0)
    def _():
        acc_ref[...] = m

    @pl.when(i > 0)
    def _():
        acc_ref[...] = jnp.minimum(acc_ref[...], m)

    @pl.when(i == _NB1 - 1)
    def _():
        min_e = acc_ref[...]
        mine_ref[...] = min_e

        d2all = min_e + aa_ref[...]
        lane = lax.broadcasted_iota(jnp.int32, (1, _N_PIX), 1)
        batch_id = lane // _HW
        cand_idx = []
        for bb_ in range(_BSZ):
            dm = jnp.where(batch_id == bb_, d2all, -_BIG)
            for _ in range(_NPB):
                am = jnp.argmax(dm, axis=1)[0]
                cand_idx.append(am)
                dm = jnp.where(lane == am, -_BIG, dm)

        cand_rows = [a2_ref[pl.ds(ix, 1), :] for ix in cand_idx]
        candT = jnp.concatenate(cand_rows, axis=0)
        cand_bf = candT.astype(jnp.bfloat16)
        aa_list = [jnp.sum(jnp.where(lane == ix, aa_ref[...], 0.0), axis=1,
                           keepdims=True) for ix in cand_idx]
        aa32 = jnp.concatenate(aa_list, axis=1)

        def p2_body(j, carry):
            top, curmin, nn = carry
            b2 = bank_ref[pl.ds(j * _BLK2, _BLK2), :]
            e2 = lax.dot_general(b2, cand_bf, (((1,), (1,)), ((), ())),
                                 preferred_element_type=jnp.float32)
            m2 = jnp.min(e2, axis=0, keepdims=True)
            am2 = jnp.argmin(e2, axis=0)[None, :] + j * _BLK2
            nn = jnp.where(m2 < curmin, am2, nn)
            curmin = jnp.minimum(curmin, m2)

            def merge(_):
                comb = jnp.concatenate([top, e2], axis=0)
                srow = lax.broadcasted_iota(jnp.int32, comb.shape, 0)
                rows = []
                for _ in range(_K):
                    v = jnp.min(comb, axis=0, keepdims=True)
                    amr = jnp.argmin(comb, axis=0)[None, :]
                    comb = jnp.where(srow == amr, _BIG, comb)
                    rows.append(v)
                rows.append(jnp.full((16 - _K, _NCAND), _BIG, jnp.float32))
                return jnp.concatenate(rows, axis=0)

            top = lax.cond(jnp.any(m2 < top[_K - 1:_K, :]), merge,
                           lambda _: top, operand=None)
            return (top, curmin, nn)

        top, _, nn = lax.fori_loop(
            0, _NB2, p2_body,
            (jnp.full((16, _NCAND), _BIG, jnp.float32),
             jnp.full((1, _NCAND), _BIG, jnp.float32),
             jnp.zeros((1, _NCAND), jnp.int32)))

        for cpos in range(_NCAND):
            row = nn[0, cpos]
            pltpu.make_async_copy(b_hbm.at[pl.ds(row, 1), :],
                                  rrows_ref.at[pl.ds(cpos, 1), :],
                                  rsems.at[cpos]).start()
        for cpos in range(_NCAND):
            row = nn[0, cpos]
            pltpu.make_async_copy(b_hbm.at[pl.ds(row, 1), :],
                                  rrows_ref.at[pl.ds(cpos, 1), :],
                                  rsems.at[cpos]).wait()
        d2x_cols = []
        for cpos in range(_NCAND):
            brow = rrows_ref[cpos:cpos + 1, :]
            arow = candT[cpos:cpos + 1, :_C]
            d2x_cols.append(jnp.sum(arow * brow + brow * brow, axis=1,
                                    keepdims=True))
        d2x = jnp.concatenate(d2x_cols, axis=1) + aa32

        conf0 = jnp.sqrt(jnp.maximum(d2x, 1e-12))
        conf_rest = jnp.sqrt(jnp.maximum(top[1:_K, :] + aa32, 1e-12))
        conf = jnp.concatenate([conf0, conf_rest], axis=0)
        ec = jnp.exp(conf)
        wgt = 1.0 - (jnp.max(ec, axis=0, keepdims=True) /
                     jnp.sum(ec, axis=0, keepdims=True))
        s32 = conf0 * wgt

        lane2 = lax.broadcasted_iota(jnp.int32, (1, _NCAND), 1)
        grp = lane2 // _NPB
        s_rows = []
        for bb_ in range(_BSZ):
            db = jnp.where(grp == bb_, d2x, -_BIG)
            vb = jnp.max(db)
            sb = jnp.max(jnp.where((grp == bb_) & (db == vb), s32, -_BIG))
            s_rows.append(jnp.full((1, 1), 1.0, jnp.float32) * sb)
        s_rows.append(jnp.zeros((4, 1), jnp.float32))
        s_ref[...] = jnp.concatenate(s_rows, axis=0)


def _mask_kern(d2_ref, a_ref, at_ref, o_ref):
    m14 = jnp.sqrt(jnp.maximum(d2_ref[0], 1e-12))
    t = jnp.dot(a_ref[...], m14, preferred_element_type=jnp.float32)
    o_ref[0] = jnp.dot(t, at_ref[...], preferred_element_type=jnp.float32)


def kernel(inputs, feature_vector):
    bsz, h, w, c = inputs.shape
    n_pix = bsz * h * w
    q = inputs.reshape(n_pix, c)
    aa = jnp.sum(q * q, axis=1)[None, :]
    a2 = jnp.concatenate([-2.0 * q, jnp.ones((n_pix, 2), jnp.float32)], axis=1)
    a_augT = a2.T

    min_e, s8 = pl.pallas_call(
        _mega_kern,
        grid=(_NB1,),
        in_specs=[
            pl.BlockSpec((c + 2, n_pix), lambda i: (0, 0)),
            pl.BlockSpec((n_pix, c + 2), lambda i: (0, 0)),
            pl.BlockSpec((1, n_pix), lambda i: (0, 0)),
            pl.BlockSpec((_BLK1, c), lambda i: (i, 0)),
            pl.BlockSpec(memory_space=pl.ANY),
        ],
        out_specs=[
            pl.BlockSpec((1, n_pix), lambda i: (0, 0)),
            pl.BlockSpec((8, 1), lambda i: (0, 0)),
        ],
        out_shape=[
            jax.ShapeDtypeStruct((1, n_pix), jnp.float32),
            jax.ShapeDtypeStruct((8, 1), jnp.float32),
        ],
        scratch_shapes=[
            pltpu.VMEM((_N_BANK, _C + 2), jnp.bfloat16),
            pltpu.VMEM((_NCAND, _C), jnp.float32),
            pltpu.VMEM((1, _N_PIX), jnp.float32),
            pltpu.SemaphoreType.DMA((_NCAND,)),
        ],
    )(a_augT, a2, aa, feature_vector, feature_vector)

    s = s8[:bsz]
    d2min = (aa + min_e).reshape(bsz, h, w)

    mask = pl.pallas_call(
        _mask_kern,
        grid=(bsz,),
        in_specs=[
            pl.BlockSpec((1, h, w), lambda i: (i, 0, 0)),
            pl.BlockSpec((224, h), lambda i: (0, 0)),
            pl.BlockSpec((h, 224), lambda i: (0, 0)),
        ],
        out_specs=pl.BlockSpec((1, 224, 224), lambda i: (i, 0, 0)),
        out_shape=jax.ShapeDtypeStruct((bsz, 224, 224), jnp.float32),
    )(d2min, jnp.asarray(_A_MAT), jnp.asarray(_AT_MAT))

    return (s, mask.reshape(bsz, 224, 224, 1))
